# baseline (speedup 1.0000x reference)
import functools

import jax
import jax.numpy as jnp
from jax.experimental import pallas as pl

B, D, C = 16384, 1024, 51


def _head_kernel(x_ref, wt_ref, b_ref, bins_ref, probs_ref, val_ref):
    logits = jnp.dot(x_ref[...], wt_ref[...], preferred_element_type=jnp.float32)
    logits = logits + b_ref[...]
    m = jnp.max(logits, axis=1, keepdims=True)
    e = jnp.exp(logits - m)
    s = jnp.sum(e, axis=1, keepdims=True)
    probs = e / s
    probs_ref[...] = probs
    val_ref[0, 0, :] = jnp.sum(probs * bins_ref[...], axis=1)


@jax.jit
def kernel(x, W, b, bins):
    blk = 1024
    nb = B // blk
    wt = W.T
    b2 = b.reshape(1, C)
    bins2 = bins.reshape(1, C)
    probs, val = pl.pallas_call(
        _head_kernel,
        grid=(nb,),
        in_specs=[
            pl.BlockSpec((blk, D), lambda i: (i, 0)),
            pl.BlockSpec((D, C), lambda i: (0, 0)),
            pl.BlockSpec((1, C), lambda i: (0, 0)),
            pl.BlockSpec((1, C), lambda i: (0, 0)),
        ],
        out_specs=[
            pl.BlockSpec((blk, C), lambda i: (i, 0)),
            pl.BlockSpec((1, 1, blk), lambda i: (i, 0, 0)),
        ],
        out_shape=[
            jax.ShapeDtypeStruct((B, C), jnp.float32),
            jax.ShapeDtypeStruct((nb, 1, blk), jnp.float32),
        ],
    )(x, wt, b2, bins2)
    return probs, val.reshape(B)


# speedup vs baseline: 1.0537x; 1.0537x over previous
import functools

import jax
import jax.numpy as jnp
from jax.experimental import pallas as pl

B, D, C = 16384, 1024, 51


def _head_kernel(x_ref, wt_ref, b_ref, red_ref, probs_ref, val_ref):
    logits = jnp.dot(x_ref[...], wt_ref[...], preferred_element_type=jnp.float32)
    logits = logits + b_ref[...]
    m = jnp.max(logits, axis=1, keepdims=True)
    e = jnp.exp(logits - m)
    r = jnp.dot(e, red_ref[...], preferred_element_type=jnp.float32)
    rinv = 1.0 / r[:, 0:1]
    probs_ref[...] = e * rinv
    val_ref[0, 0, :] = (r[:, 1] * rinv[:, 0])


@jax.jit
def kernel(x, W, b, bins):
    blk = 1024
    nb = B // blk
    wt = W.T
    b2 = b.reshape(1, C)
    red = jnp.stack([jnp.ones((C,), jnp.float32), bins], axis=1)
    probs, val = pl.pallas_call(
        _head_kernel,
        grid=(nb,),
        in_specs=[
            pl.BlockSpec((blk, D), lambda i: (i, 0)),
            pl.BlockSpec((D, C), lambda i: (0, 0)),
            pl.BlockSpec((1, C), lambda i: (0, 0)),
            pl.BlockSpec((C, 2), lambda i: (0, 0)),
        ],
        out_specs=[
            pl.BlockSpec((blk, C), lambda i: (i, 0)),
            pl.BlockSpec((1, 1, blk), lambda i: (i, 0, 0)),
        ],
        out_shape=[
            jax.ShapeDtypeStruct((B, C), jnp.float32),
            jax.ShapeDtypeStruct((nb, 1, blk), jnp.float32),
        ],
    )(x, wt, b2, red)
    return probs, val.reshape(B)
